# Initial kernel scaffold; baseline (speedup 1.0000x reference)
import jax
import jax.numpy as jnp
from jax.experimental import pallas as pl


def kernel(x, table):
    raise NotImplementedError("write your pallas kernel here")


# speedup vs baseline: 1.7338x; 1.7338x over previous
import functools

import jax
import jax.numpy as jnp
from jax import lax
from jax.experimental import pallas as pl
from jax.experimental.pallas import tpu as pltpu
from jax.experimental.pallas import tpu_sc as plsc


@functools.cache
def _make_lookup(B, V, D, NC, NS):
    NW = NC * NS
    b_per_w = B // NW
    C = 64
    n_chunks = b_per_w // C
    mesh = plsc.VectorSubcoreMesh(core_axis_name="c", subcore_axis_name="s")

    @functools.partial(
        pl.kernel,
        out_type=jax.ShapeDtypeStruct((B, D), jnp.float32),
        mesh=mesh,
        scratch_types=[
            pltpu.VMEM((b_per_w,), jnp.int32),
            pltpu.VMEM((C, D), jnp.float32),
            pltpu.SemaphoreType.DMA,
        ],
    )
    def lookup(idx_hbm, table_hbm, out_hbm, idx_v, rows_v, sem):
        wid = lax.axis_index("s") * NC + lax.axis_index("c")
        base = wid * b_per_w
        pltpu.sync_copy(idx_hbm.at[pl.ds(base, b_per_w)], idx_v)

        def body(i, carry):
            off = i * C
            pltpu.async_copy(
                table_hbm.at[idx_v.at[pl.ds(off, C)]], rows_v, sem
            ).wait()
            pltpu.sync_copy(rows_v, out_hbm.at[pl.ds(base + off, C)])
            return carry

        lax.fori_loop(0, n_chunks, body, 0)

    return lookup


def kernel(x, table):
    Bt, L = x.shape
    V, D = table.shape
    B = Bt * L
    info = plsc.get_sparse_core_info()
    lookup = _make_lookup(B, V, D, info.num_cores, info.num_subcores)
    out = lookup(x.reshape(B), table)
    return out.reshape(Bt, L, D)


# speedup vs baseline: 1.9623x; 1.1318x over previous
import functools

import jax
import jax.numpy as jnp
from jax import lax
from jax.experimental import pallas as pl
from jax.experimental.pallas import tpu as pltpu
from jax.experimental.pallas import tpu_sc as plsc


@functools.cache
def _make_lookup(B, V, D, NC, NS):
    NW = NC * NS
    b_per_w = B // NW
    C = 64
    n_chunks = b_per_w // C
    n_pairs = n_chunks // 2
    mesh = plsc.VectorSubcoreMesh(core_axis_name="c", subcore_axis_name="s")

    @functools.partial(
        pl.kernel,
        out_type=jax.ShapeDtypeStruct((B, D), jnp.float32),
        mesh=mesh,
        scratch_types=[
            pltpu.VMEM((b_per_w,), jnp.int32),
            pltpu.VMEM((C, D), jnp.float32),
            pltpu.VMEM((C, D), jnp.float32),
            pltpu.SemaphoreType.DMA,
            pltpu.SemaphoreType.DMA,
            pltpu.SemaphoreType.DMA,
            pltpu.SemaphoreType.DMA,
        ],
    )
    def lookup(idx_hbm, table_hbm, out_hbm, idx_v, rows0, rows1,
               g0, g1, s0, s1):
        wid = lax.axis_index("s") * NC + lax.axis_index("c")
        base = wid * b_per_w
        pltpu.sync_copy(idx_hbm.at[pl.ds(base, b_per_w)], idx_v)

        bufs = (rows0, rows1)
        gsems = (g0, g1)
        ssems = (s0, s1)

        def gather(i, b):
            pltpu.async_copy(
                table_hbm.at[idx_v.at[pl.ds(i * C, C)]], bufs[b], gsems[b])

        def wait_gather(b):
            pltpu.make_async_copy(
                table_hbm.at[idx_v.at[pl.ds(0, C)]], bufs[b], gsems[b]).wait()

        def store(i, b):
            pltpu.async_copy(
                bufs[b], out_hbm.at[pl.ds(base + i * C, C)], ssems[b])

        def wait_store(b):
            pltpu.make_async_copy(
                bufs[b], out_hbm.at[pl.ds(base, C)], ssems[b]).wait()

        gather(0, 0)

        def body(j, carry):
            for b in range(2):
                i = 2 * j + b
                nb = 1 - b

                @pl.when(i > 0)
                def _():
                    wait_store(nb)

                @pl.when(i + 1 < n_chunks)
                def _():
                    gather(i + 1, nb)

                wait_gather(b)
                store(i, b)
            return carry

        lax.fori_loop(0, n_pairs, body, 0)
        wait_store((n_chunks - 1) % 2)

    return lookup


def kernel(x, table):
    Bt, L = x.shape
    V, D = table.shape
    B = Bt * L
    info = plsc.get_sparse_core_info()
    lookup = _make_lookup(B, V, D, info.num_cores, info.num_subcores)
    out = lookup(x.reshape(B), table)
    return out.reshape(Bt, L, D)


# speedup vs baseline: 1.9644x; 1.0011x over previous
import functools

import jax
import jax.numpy as jnp
from jax import lax
from jax.experimental import pallas as pl
from jax.experimental.pallas import tpu as pltpu
from jax.experimental.pallas import tpu_sc as plsc


@functools.cache
def _make_lookup(B, V, D, NC, NS):
    NW = NC * NS
    b_per_w = B // NW
    C = 80
    n_chunks = b_per_w // C
    n_pairs = n_chunks // 2
    mesh = plsc.VectorSubcoreMesh(core_axis_name="c", subcore_axis_name="s")

    @functools.partial(
        pl.kernel,
        out_type=jax.ShapeDtypeStruct((B, D), jnp.float32),
        mesh=mesh,
        scratch_types=[
            pltpu.VMEM((C,), jnp.int32),
            pltpu.VMEM((C,), jnp.int32),
            pltpu.VMEM((C, D), jnp.float32),
            pltpu.VMEM((C, D), jnp.float32),
            pltpu.SemaphoreType.DMA,
            pltpu.SemaphoreType.DMA,
            pltpu.SemaphoreType.DMA,
            pltpu.SemaphoreType.DMA,
            pltpu.SemaphoreType.DMA,
            pltpu.SemaphoreType.DMA,
        ],
    )
    def lookup(idx_hbm, table_hbm, out_hbm, idx0, idx1, rows0, rows1,
               i0, i1, g0, g1, s0, s1):
        wid = lax.axis_index("s") * NC + lax.axis_index("c")
        base = wid * b_per_w

        ibufs = (idx0, idx1)
        bufs = (rows0, rows1)
        isems = (i0, i1)
        gsems = (g0, g1)
        ssems = (s0, s1)

        def idx_load(i, b):
            pltpu.async_copy(
                idx_hbm.at[pl.ds(base + i * C, C)], ibufs[b], isems[b])

        def wait_idx(b):
            pltpu.make_async_copy(
                idx_hbm.at[pl.ds(base, C)], ibufs[b], isems[b]).wait()

        def gather(b):
            pltpu.async_copy(table_hbm.at[ibufs[b]], bufs[b], gsems[b])

        def wait_gather(b):
            pltpu.make_async_copy(
                table_hbm.at[ibufs[b]], bufs[b], gsems[b]).wait()

        def store(i, b):
            pltpu.async_copy(
                bufs[b], out_hbm.at[pl.ds(base + i * C, C)], ssems[b])

        def wait_store(b):
            pltpu.make_async_copy(
                bufs[b], out_hbm.at[pl.ds(base, C)], ssems[b]).wait()

        idx_load(0, 0)
        wait_idx(0)
        gather(0)
        idx_load(1, 1)

        def body(j, carry):
            for b in range(2):
                i = 2 * j + b
                nb = 1 - b

                @pl.when(i > 0)
                def _():
                    wait_store(nb)

                @pl.when(i + 1 < n_chunks)
                def _():
                    wait_idx(nb)
                    gather(nb)

                wait_gather(b)

                @pl.when(i + 2 < n_chunks)
                def _():
                    idx_load(i + 2, b)

                store(i, b)
            return carry

        lax.fori_loop(0, n_pairs, body, 0)
        wait_store((n_chunks - 1) % 2)

    return lookup


def kernel(x, table):
    Bt, L = x.shape
    V, D = table.shape
    B = Bt * L
    info = plsc.get_sparse_core_info()
    lookup = _make_lookup(B, V, D, info.num_cores, info.num_subcores)
    out = lookup(x.reshape(B), table)
    return out.reshape(Bt, L, D)
